# baseline (speedup 1.0000x reference)
import functools

import jax
import jax.numpy as jnp
from jax import lax
from jax.experimental import pallas as pl
from jax.experimental.pallas import tpu as pltpu
from jax.experimental.pallas import tpu_sc as plsc

NC = 2
NS = 16
NW = NC * NS
CHUNK = 96


def _make_deg(n_pad, e_pad):
    per_tile = e_pad // NW
    steps = per_tile // 16
    mesh = plsc.VectorSubcoreMesh(core_axis_name="c", subcore_axis_name="s")

    @functools.partial(
        pl.kernel,
        out_type=jax.ShapeDtypeStruct((NW * n_pad,), jnp.float32),
        mesh=mesh,
        compiler_params=pltpu.CompilerParams(needs_layout_passes=False),
        scratch_types=[
            pltpu.VMEM((n_pad,), jnp.float32),
            pltpu.VMEM((per_tile,), jnp.int32),
            pltpu.VMEM((per_tile,), jnp.float32),
        ],
    )
    def deg_kernel(dst_hbm, ew_hbm, out_hbm, table, dst_v, ew_v):
        cid = lax.axis_index("c")
        sid = lax.axis_index("s")
        wid = cid * NS + sid
        base = wid * per_tile
        pltpu.sync_copy(dst_hbm.at[pl.ds(base, per_tile)], dst_v)
        pltpu.sync_copy(ew_hbm.at[pl.ds(base, per_tile)], ew_v)
        zero16 = jnp.zeros((16,), jnp.float32)

        def _zero(r, c):
            table[pl.ds(r * 16, 16)] = zero16
            return c

        lax.fori_loop(0, n_pad // 16, _zero, 0)

        def _step(i, c):
            dv = dst_v[pl.ds(i * 16, 16)]
            wv = ew_v[pl.ds(i * 16, 16)]
            plsc.addupdate_scatter(table, [dv], wv)
            return c

        lax.fori_loop(0, steps, _step, 0)
        pltpu.sync_copy(table, out_hbm.at[pl.ds(wid * n_pad, n_pad)])

    return deg_kernel


def _make_scatter(n_pad, c0, c1, d):
    assert c0 % 3 == 0 and c1 % 3 == 0
    cap0, cap1 = c0 * CHUNK, c1 * CHUNK
    cap_max = max(cap0, cap1)
    rows_per_sub = n_pad // NS
    mesh = plsc.VectorSubcoreMesh(core_axis_name="c", subcore_axis_name="s")

    @functools.partial(
        pl.kernel,
        out_type=jax.ShapeDtypeStruct((NC, n_pad, d), jnp.float32),
        mesh=mesh,
        compiler_params=pltpu.CompilerParams(needs_layout_passes=False),
        scratch_types=[
            pltpu.VMEM_SHARED((n_pad, d), jnp.float32),
            pltpu.VMEM((cap_max,), jnp.int32),
            pltpu.VMEM((CHUNK, d), jnp.float32),
            pltpu.VMEM((CHUNK, d), jnp.float32),
            pltpu.VMEM((CHUNK, d), jnp.float32),
            pltpu.VMEM((CHUNK,), jnp.int32),
            pltpu.VMEM((CHUNK,), jnp.int32),
            pltpu.VMEM((CHUNK,), jnp.int32),
            pltpu.VMEM((CHUNK,), jnp.int32),
            pltpu.VMEM((CHUNK,), jnp.int32),
            pltpu.VMEM((CHUNK,), jnp.int32),
            pltpu.VMEM((CHUNK,), jnp.float32),
            pltpu.VMEM((CHUNK,), jnp.float32),
            pltpu.VMEM((CHUNK,), jnp.float32),
            pltpu.SemaphoreType.DMA,
            pltpu.SemaphoreType.DMA,
            pltpu.SemaphoreType.DMA,
            pltpu.SemaphoreType.DMA,
            pltpu.SemaphoreType.DMA,
            pltpu.SemaphoreType.DMA,
            pltpu.SemaphoreType.DMA,
            pltpu.SemaphoreType.DMA,
            pltpu.SemaphoreType.DMA,
        ],
    )
    def scatter_kernel(g_hbm, pk_hbm, ew_hbm, out_hbm,
                       acc, pk_all, r0b, r1b, r2b,
                       sb0, sb1, sb2, db0, db1, db2, eb0, eb1, eb2,
                       g0, g1, g2, s0, s1, s2, e0, e1, e2):
        cid = lax.axis_index("c")
        sid = lax.axis_index("s")
        r0 = sid * rows_per_sub
        bufs = (r0b, r1b, r2b)
        srcb = (sb0, sb1, sb2)
        dstb = (db0, db1, db2)
        ewb = (eb0, eb1, eb2)
        gsems = (g0, g1, g2)
        ssems = (s0, s1, s2)
        esems = (e0, e1, e2)

        base = jnp.where(cid == 0, sid * cap0, NS * cap0 + sid * cap1)
        chunks_t = jnp.where(cid == 0, c0, c1)

        @pl.when(cid == 0)
        def _():
            pltpu.sync_copy(pk_hbm.at[pl.ds(sid * cap0, cap0)],
                            pk_all.at[pl.ds(0, cap0)])

        @pl.when(cid == 1)
        def _():
            pltpu.sync_copy(pk_hbm.at[pl.ds(NS * cap0 + sid * cap1, cap1)],
                            pk_all.at[pl.ds(0, cap1)])

        zero16 = jnp.zeros((16,), jnp.float32)

        def _zrow(r, c):
            def _zcol(cb, cc):
                r0b[r, pl.ds(cb * 16, 16)] = zero16
                return cc
            return lax.fori_loop(0, d // 16, _zcol, c, unroll=True)

        lax.fori_loop(0, CHUNK, _zrow, 0)
        for off in range(0, rows_per_sub, CHUNK):
            sz = min(CHUNK, rows_per_sub - off)
            pltpu.sync_copy(r0b.at[pl.ds(0, sz)],
                            acc.at[pl.ds(r0 + off, sz)])
        plsc.subcore_barrier()

        def _unpack(j, b):
            def _u(q, c):
                v = pk_all[pl.ds(j * CHUNK + q * 16, 16)]
                srcb[b][pl.ds(q * 16, 16)] = v & 0xFFFF
                dstb[b][pl.ds(q * 16, 16)] = lax.shift_right_logical(v, 16)
                return c
            lax.fori_loop(0, CHUNK // 16, _u, 0, unroll=True)

        def _load_ew(j, b, sem):
            pltpu.async_copy(ew_hbm.at[pl.ds(base + j * CHUNK, CHUNK)],
                             ewb[b], sem)

        def _scale(buf, ew_s):
            def _srow(r, cc):
                w = plsc.load_gather(ew_s, [jnp.full((16,), 0, jnp.int32) + r])


                def _scol(cb, c3):
                    sl = pl.ds(cb * 16, 16)
                    buf[r, sl] = buf[r, sl] * w
                    return c3

                return lax.fori_loop(0, d // 16, _scol, cc, unroll=True)

            lax.fori_loop(0, CHUNK, _srow, 0, unroll=8)

        _unpack(0, 0)
        _unpack(1, 1)
        pltpu.async_copy(g_hbm.at[sb0], r0b, g0)
        pltpu.async_copy(g_hbm.at[sb1], r1b, g1)
        _load_ew(0, 0, e0)
        _load_ew(1, 1, e1)

        def _triplet(gi, c):
            for b in range(3):
                j = gi * 3 + b
                bn = (b + 2) % 3
                pltpu.make_async_copy(g_hbm.at[srcb[b]], bufs[b],
                                      gsems[b]).wait()
                pltpu.make_async_copy(ew_hbm.at[pl.ds(0, CHUNK)], ewb[b],
                                      esems[b]).wait()
                _scale(bufs[b], ewb[b])

                @pl.when(j > 0)
                def _():
                    pltpu.make_async_copy(bufs[bn], acc.at[dstb[bn]],
                                          ssems[bn]).wait()

                @pl.when(j + 2 < chunks_t)
                def _():
                    _unpack(j + 2, bn)
                    pltpu.async_copy(g_hbm.at[srcb[bn]], bufs[bn], gsems[bn])
                    _load_ew(j + 2, bn, esems[bn])

                pltpu.async_copy(bufs[b], acc.at[dstb[b]], ssems[b], add=True)
            return c

        lax.fori_loop(0, chunks_t // 3, _triplet, 0)
        pltpu.make_async_copy(r2b, acc.at[db2], s2).wait()

        plsc.subcore_barrier()
        pltpu.sync_copy(acc.at[pl.ds(r0, rows_per_sub)],
                        out_hbm.at[cid, pl.ds(r0, rows_per_sub)])

    return scatter_kernel


def _dis_from_partials(degp):
    nw, n = degp.shape

    def body(dp_ref, out_ref):
        s = jnp.sum(dp_ref[...], axis=0) + 1.0
        out_ref[...] = jnp.where(s > 0, lax.rsqrt(s), 0.0)

    return pl.pallas_call(
        body,
        out_shape=jax.ShapeDtypeStruct((n,), jnp.float32),
    )(degp)


_ROWS_BLK = 1000


def _mm_scale(x, w, dis2):
    n, d = x.shape

    def body(x_ref, w_ref, dis_ref, out_ref):
        out_ref[...] = dis_ref[...] * jnp.dot(
            x_ref[...], w_ref[...],
            preferred_element_type=jnp.float32,
            precision=lax.Precision.HIGHEST)

    return pl.pallas_call(
        body,
        grid=(n // _ROWS_BLK,),
        in_specs=[
            pl.BlockSpec((_ROWS_BLK, d), lambda i: (i, 0)),
            pl.BlockSpec((d, d), lambda i: (0, 0)),
            pl.BlockSpec((_ROWS_BLK, 1), lambda i: (i, 0)),
        ],
        out_specs=pl.BlockSpec((_ROWS_BLK, d), lambda i: (i, 0)),
        out_shape=jax.ShapeDtypeStruct((n, d), jnp.float32),
    )(x, w, dis2)


def _fused_layer(p, g, dis2, b2, w):
    n, d = g.shape

    def body(p_ref, g_ref, dis_ref, b_ref, w_ref, out_ref):
        s = p_ref[0] + p_ref[1] + g_ref[...]
        h = jnp.maximum(dis_ref[...] * s + b_ref[...], 0.0)
        out_ref[...] = dis_ref[...] * jnp.dot(
            h, w_ref[...],
            preferred_element_type=jnp.float32,
            precision=lax.Precision.HIGHEST)

    return pl.pallas_call(
        body,
        grid=(n // _ROWS_BLK,),
        in_specs=[
            pl.BlockSpec((NC, _ROWS_BLK, d), lambda i: (0, i, 0)),
            pl.BlockSpec((_ROWS_BLK, d), lambda i: (i, 0)),
            pl.BlockSpec((_ROWS_BLK, 1), lambda i: (i, 0)),
            pl.BlockSpec((1, d), lambda i: (0, 0)),
            pl.BlockSpec((d, d), lambda i: (0, 0)),
        ],
        out_specs=pl.BlockSpec((_ROWS_BLK, d), lambda i: (i, 0)),
        out_shape=jax.ShapeDtypeStruct((n, d), jnp.float32),
    )(p, g, dis2, b2, w)


def _finish(p, g, dis2, b2):
    n, d = g.shape

    def body(p_ref, g_ref, dis_ref, b_ref, out_ref):
        s = p_ref[0] + p_ref[1] + g_ref[...]
        out_ref[...] = dis_ref[...] * s + b_ref[...]

    return pl.pallas_call(
        body,
        grid=(n // _ROWS_BLK,),
        in_specs=[
            pl.BlockSpec((NC, _ROWS_BLK, d), lambda i: (0, i, 0)),
            pl.BlockSpec((_ROWS_BLK, d), lambda i: (i, 0)),
            pl.BlockSpec((_ROWS_BLK, 1), lambda i: (i, 0)),
            pl.BlockSpec((1, d), lambda i: (0, 0)),
        ],
        out_specs=pl.BlockSpec((_ROWS_BLK, d), lambda i: (i, 0)),
        out_shape=jax.ShapeDtypeStruct((n, d), jnp.float32),
    )(p, g, dis2, b2)


_CORE0_FRAC = 0.405


def _edge_layout(e):
    import numpy as np
    e0 = (int(e * _CORE0_FRAC) // NS) * NS
    r0 = e0 // NS
    e1 = e - e0
    r1, rem = divmod(e1, NS)

    def _cap(r):
        ch = -(-r // CHUNK)
        ch = ((ch + 2) // 3) * 3
        return max(ch, 3)

    c0, c1 = _cap(r0), _cap(r1 + (1 if rem else 0))
    cap0, cap1 = c0 * CHUNK, c1 * CHUNK
    perm = np.full(NS * cap0 + NS * cap1, e, dtype=np.int32)
    idx = 0
    for t in range(NS):
        perm[t * cap0:t * cap0 + r0] = np.arange(idx, idx + r0)
        idx += r0
    for t in range(NS):
        cnt = r1 + (1 if t < rem else 0)
        b = NS * cap0 + t * cap1
        perm[b:b + cnt] = np.arange(idx, idx + cnt)
        idx += cnt
    assert idx == e
    return perm, c0, c1


def kernel(x, edge_index, edge_attr, W1, b1, W2, b2, W3, b3, W4, b4, W5, b5):
    n, d = x.shape
    e = edge_index.shape[1]
    src = edge_index[0].astype(jnp.int32)
    dst = edge_index[1].astype(jnp.int32)
    ew = edge_attr.astype(jnp.float32)

    grp = NW * 16
    e_pad = ((e + grp - 1) // grp) * grp
    pad = e_pad - e
    dst_p = jnp.pad(dst, (0, pad))
    ew_p = jnp.pad(ew, (0, pad))

    perm, c0, c1 = _edge_layout(e)
    perm = jnp.asarray(perm)
    src_l = jnp.pad(src, (0, 1))[perm]
    dst_l = jnp.pad(dst, (0, 1))[perm]
    ew_l = jnp.pad(ew, (0, 1))[perm]
    pk_l = src_l | (dst_l << 16)

    ngrp = NS * 8
    n_pad = ((n + ngrp - 1) // ngrp) * ngrp

    deg_call = _make_deg(n_pad, e_pad)
    scat_call = _make_scatter(n_pad, c0, c1, d)

    degp = deg_call(dst_p, ew_p)
    dis = _dis_from_partials(degp.reshape(NW, n_pad))
    dis2 = dis[:n].reshape(n, 1)

    g = _mm_scale(x, W1, dis2)
    for b_i, w_next in ((b1, W2), (b2, W3), (b3, W4), (b4, W5)):
        p = scat_call(g, pk_l, ew_l)
        g = _fused_layer(p, g, dis2, b_i.reshape(1, d), w_next)
    p = scat_call(g, pk_l, ew_l)
    return _finish(p, g, dis2, b5.reshape(1, d))


# speedup vs baseline: 1.6339x; 1.6339x over previous
import functools

import jax
import jax.numpy as jnp
from jax import lax
from jax.experimental import pallas as pl
from jax.experimental.pallas import tpu as pltpu
from jax.experimental.pallas import tpu_sc as plsc

NC = 2
NS = 16
NW = NC * NS
CHUNK = 64


def _make_deg(n_pad, e_pad):
    per_tile = e_pad // NW
    steps = per_tile // 16
    mesh = plsc.VectorSubcoreMesh(core_axis_name="c", subcore_axis_name="s")

    @functools.partial(
        pl.kernel,
        out_type=jax.ShapeDtypeStruct((NW * n_pad,), jnp.float32),
        mesh=mesh,
        compiler_params=pltpu.CompilerParams(needs_layout_passes=False),
        scratch_types=[
            pltpu.VMEM((n_pad,), jnp.float32),
            pltpu.VMEM((per_tile,), jnp.int32),
            pltpu.VMEM((per_tile,), jnp.float32),
        ],
    )
    def deg_kernel(dst_hbm, ew_hbm, out_hbm, table, dst_v, ew_v):
        cid = lax.axis_index("c")
        sid = lax.axis_index("s")
        wid = cid * NS + sid
        base = wid * per_tile
        pltpu.sync_copy(dst_hbm.at[pl.ds(base, per_tile)], dst_v)
        pltpu.sync_copy(ew_hbm.at[pl.ds(base, per_tile)], ew_v)
        zero16 = jnp.zeros((16,), jnp.float32)

        def _zero(r, c):
            table[pl.ds(r * 16, 16)] = zero16
            return c

        lax.fori_loop(0, n_pad // 16, _zero, 0)

        def _step(i, c):
            dv = dst_v[pl.ds(i * 16, 16)]
            wv = ew_v[pl.ds(i * 16, 16)]
            plsc.addupdate_scatter(table, [dv], wv)
            return c

        lax.fori_loop(0, steps, _step, 0)
        pltpu.sync_copy(table, out_hbm.at[pl.ds(wid * n_pad, n_pad)])

    return deg_kernel


def _make_scatter(n_pad, c0, c1, d):
    assert c0 % 3 == 0 and c1 % 3 == 0
    cap0, cap1 = c0 * CHUNK, c1 * CHUNK
    cap_max = max(cap0, cap1)
    rows_per_sub = n_pad // NS
    mesh = plsc.VectorSubcoreMesh(core_axis_name="c", subcore_axis_name="s")

    @functools.partial(
        pl.kernel,
        out_type=jax.ShapeDtypeStruct((NC, n_pad, d), jnp.float32),
        mesh=mesh,
        compiler_params=pltpu.CompilerParams(needs_layout_passes=False),
        scratch_types=[
            pltpu.VMEM_SHARED((n_pad, d), jnp.float32),
            pltpu.VMEM((cap_max,), jnp.int32),
            pltpu.VMEM((CHUNK, d), jnp.float32),
            pltpu.VMEM((CHUNK, d), jnp.float32),
            pltpu.VMEM((CHUNK, d), jnp.float32),
            pltpu.VMEM((CHUNK,), jnp.int32),
            pltpu.VMEM((CHUNK,), jnp.int32),
            pltpu.VMEM((CHUNK,), jnp.int32),
            pltpu.VMEM((CHUNK,), jnp.int32),
            pltpu.VMEM((CHUNK,), jnp.int32),
            pltpu.VMEM((CHUNK,), jnp.int32),
            pltpu.VMEM((CHUNK,), jnp.float32),
            pltpu.VMEM((CHUNK,), jnp.float32),
            pltpu.VMEM((CHUNK,), jnp.float32),
            pltpu.SemaphoreType.DMA,
            pltpu.SemaphoreType.DMA,
            pltpu.SemaphoreType.DMA,
            pltpu.SemaphoreType.DMA,
            pltpu.SemaphoreType.DMA,
            pltpu.SemaphoreType.DMA,
            pltpu.SemaphoreType.DMA,
            pltpu.SemaphoreType.DMA,
            pltpu.SemaphoreType.DMA,
        ],
    )
    def scatter_kernel(g_hbm, pk_hbm, ew_hbm, out_hbm,
                       acc, pk_all, r0b, r1b, r2b,
                       sb0, sb1, sb2, db0, db1, db2, eb0, eb1, eb2,
                       g0, g1, g2, s0, s1, s2, e0, e1, e2):
        cid = lax.axis_index("c")
        sid = lax.axis_index("s")
        r0 = sid * rows_per_sub
        bufs = (r0b, r1b, r2b)
        srcb = (sb0, sb1, sb2)
        dstb = (db0, db1, db2)
        ewb = (eb0, eb1, eb2)
        gsems = (g0, g1, g2)
        ssems = (s0, s1, s2)
        esems = (e0, e1, e2)

        base = jnp.where(cid == 0, sid * cap0, NS * cap0 + sid * cap1)
        chunks_t = jnp.where(cid == 0, c0, c1)

        @pl.when(cid == 0)
        def _():
            pltpu.sync_copy(pk_hbm.at[pl.ds(sid * cap0, cap0)],
                            pk_all.at[pl.ds(0, cap0)])

        @pl.when(cid == 1)
        def _():
            pltpu.sync_copy(pk_hbm.at[pl.ds(NS * cap0 + sid * cap1, cap1)],
                            pk_all.at[pl.ds(0, cap1)])

        zero16 = jnp.zeros((16,), jnp.float32)

        def _zrow(r, c):
            def _zcol(cb, cc):
                r0b[r, pl.ds(cb * 16, 16)] = zero16
                return cc
            return lax.fori_loop(0, d // 16, _zcol, c, unroll=True)

        lax.fori_loop(0, CHUNK, _zrow, 0)
        for off in range(0, rows_per_sub, CHUNK):
            sz = min(CHUNK, rows_per_sub - off)
            pltpu.sync_copy(r0b.at[pl.ds(0, sz)],
                            acc.at[pl.ds(r0 + off, sz)])
        plsc.subcore_barrier()

        def _unpack(j, b):
            def _u(q, c):
                v = pk_all[pl.ds(j * CHUNK + q * 16, 16)]
                srcb[b][pl.ds(q * 16, 16)] = v & 0xFFFF
                dstb[b][pl.ds(q * 16, 16)] = lax.shift_right_logical(v, 16)
                return c
            lax.fori_loop(0, CHUNK // 16, _u, 0, unroll=True)

        def _load_ew(j, b, sem):
            pltpu.async_copy(ew_hbm.at[pl.ds(base + j * CHUNK, CHUNK)],
                             ewb[b], sem)

        def _scale(buf, ew_s):
            def _srow(r, cc):
                w = plsc.load_gather(ew_s, [jnp.full((16,), r, jnp.int32)])

                def _scol(cb, c3):
                    sl = pl.ds(cb * 16, 16)
                    buf[r, sl] = buf[r, sl] * w
                    return c3

                return lax.fori_loop(0, d // 16, _scol, cc, unroll=True)

            lax.fori_loop(0, CHUNK, _srow, 0, unroll=4)

        _unpack(0, 0)
        _unpack(1, 1)
        pltpu.async_copy(g_hbm.at[sb0], r0b, g0)
        pltpu.async_copy(g_hbm.at[sb1], r1b, g1)
        _load_ew(0, 0, e0)
        _load_ew(1, 1, e1)

        def _triplet(gi, c):
            for b in range(3):
                j = gi * 3 + b
                bn = (b + 2) % 3
                pltpu.make_async_copy(g_hbm.at[srcb[b]], bufs[b],
                                      gsems[b]).wait()
                pltpu.make_async_copy(ew_hbm.at[pl.ds(0, CHUNK)], ewb[b],
                                      esems[b]).wait()
                _scale(bufs[b], ewb[b])

                @pl.when(j > 0)
                def _():
                    pltpu.make_async_copy(bufs[bn], acc.at[dstb[bn]],
                                          ssems[bn]).wait()

                @pl.when(j + 2 < chunks_t)
                def _():
                    _unpack(j + 2, bn)
                    pltpu.async_copy(g_hbm.at[srcb[bn]], bufs[bn], gsems[bn])
                    _load_ew(j + 2, bn, esems[bn])

                pltpu.async_copy(bufs[b], acc.at[dstb[b]], ssems[b], add=True)
            return c

        lax.fori_loop(0, chunks_t // 3, _triplet, 0)
        pltpu.make_async_copy(r2b, acc.at[db2], s2).wait()

        plsc.subcore_barrier()
        pltpu.sync_copy(acc.at[pl.ds(r0, rows_per_sub)],
                        out_hbm.at[cid, pl.ds(r0, rows_per_sub)])

    return scatter_kernel


def _dis_from_partials(degp):
    nw, n = degp.shape

    def body(dp_ref, out_ref):
        s = jnp.sum(dp_ref[...], axis=0) + 1.0
        out_ref[...] = jnp.where(s > 0, lax.rsqrt(s), 0.0)

    return pl.pallas_call(
        body,
        out_shape=jax.ShapeDtypeStruct((n,), jnp.float32),
    )(degp)


_ROWS_BLK = 1000


def _mm_scale(x, w, dis2):
    n, d = x.shape

    def body(x_ref, w_ref, dis_ref, out_ref):
        out_ref[...] = dis_ref[...] * jnp.dot(
            x_ref[...], w_ref[...],
            preferred_element_type=jnp.float32,
            precision=lax.Precision.HIGHEST)

    return pl.pallas_call(
        body,
        grid=(n // _ROWS_BLK,),
        in_specs=[
            pl.BlockSpec((_ROWS_BLK, d), lambda i: (i, 0)),
            pl.BlockSpec((d, d), lambda i: (0, 0)),
            pl.BlockSpec((_ROWS_BLK, 1), lambda i: (i, 0)),
        ],
        out_specs=pl.BlockSpec((_ROWS_BLK, d), lambda i: (i, 0)),
        out_shape=jax.ShapeDtypeStruct((n, d), jnp.float32),
    )(x, w, dis2)


def _fused_layer(p, g, dis2, b2, w):
    n, d = g.shape

    def body(p_ref, g_ref, dis_ref, b_ref, w_ref, out_ref):
        s = p_ref[0] + p_ref[1] + g_ref[...]
        h = jnp.maximum(dis_ref[...] * s + b_ref[...], 0.0)
        out_ref[...] = dis_ref[...] * jnp.dot(
            h, w_ref[...],
            preferred_element_type=jnp.float32,
            precision=lax.Precision.HIGHEST)

    return pl.pallas_call(
        body,
        grid=(n // _ROWS_BLK,),
        in_specs=[
            pl.BlockSpec((NC, _ROWS_BLK, d), lambda i: (0, i, 0)),
            pl.BlockSpec((_ROWS_BLK, d), lambda i: (i, 0)),
            pl.BlockSpec((_ROWS_BLK, 1), lambda i: (i, 0)),
            pl.BlockSpec((1, d), lambda i: (0, 0)),
            pl.BlockSpec((d, d), lambda i: (0, 0)),
        ],
        out_specs=pl.BlockSpec((_ROWS_BLK, d), lambda i: (i, 0)),
        out_shape=jax.ShapeDtypeStruct((n, d), jnp.float32),
    )(p, g, dis2, b2, w)


def _finish(p, g, dis2, b2):
    n, d = g.shape

    def body(p_ref, g_ref, dis_ref, b_ref, out_ref):
        s = p_ref[0] + p_ref[1] + g_ref[...]
        out_ref[...] = dis_ref[...] * s + b_ref[...]

    return pl.pallas_call(
        body,
        grid=(n // _ROWS_BLK,),
        in_specs=[
            pl.BlockSpec((NC, _ROWS_BLK, d), lambda i: (0, i, 0)),
            pl.BlockSpec((_ROWS_BLK, d), lambda i: (i, 0)),
            pl.BlockSpec((_ROWS_BLK, 1), lambda i: (i, 0)),
            pl.BlockSpec((1, d), lambda i: (0, 0)),
        ],
        out_specs=pl.BlockSpec((_ROWS_BLK, d), lambda i: (i, 0)),
        out_shape=jax.ShapeDtypeStruct((n, d), jnp.float32),
    )(p, g, dis2, b2)


_CORE0_FRAC = 0.405


def _edge_layout(e):
    import numpy as np
    e0 = (int(e * _CORE0_FRAC) // NS) * NS
    r0 = e0 // NS
    e1 = e - e0
    r1, rem = divmod(e1, NS)

    def _cap(r):
        ch = -(-r // CHUNK)
        ch = ((ch + 2) // 3) * 3
        return max(ch, 3)

    c0, c1 = _cap(r0), _cap(r1 + (1 if rem else 0))
    cap0, cap1 = c0 * CHUNK, c1 * CHUNK
    perm = np.full(NS * cap0 + NS * cap1, e, dtype=np.int32)
    idx = 0
    for t in range(NS):
        perm[t * cap0:t * cap0 + r0] = np.arange(idx, idx + r0)
        idx += r0
    for t in range(NS):
        cnt = r1 + (1 if t < rem else 0)
        b = NS * cap0 + t * cap1
        perm[b:b + cnt] = np.arange(idx, idx + cnt)
        idx += cnt
    assert idx == e
    return perm, c0, c1


def kernel(x, edge_index, edge_attr, W1, b1, W2, b2, W3, b3, W4, b4, W5, b5):
    n, d = x.shape
    e = edge_index.shape[1]
    src = edge_index[0].astype(jnp.int32)
    dst = edge_index[1].astype(jnp.int32)
    ew = edge_attr.astype(jnp.float32)

    grp = NW * 16
    e_pad = ((e + grp - 1) // grp) * grp
    pad = e_pad - e
    dst_p = jnp.pad(dst, (0, pad))
    ew_p = jnp.pad(ew, (0, pad))

    perm, c0, c1 = _edge_layout(e)
    perm = jnp.asarray(perm)
    src_l = jnp.pad(src, (0, 1))[perm]
    dst_l = jnp.pad(dst, (0, 1))[perm]
    ew_l = jnp.pad(ew, (0, 1))[perm]
    pk_l = src_l | (dst_l << 16)

    ngrp = NS * 128
    n_pad = ((n + ngrp - 1) // ngrp) * ngrp

    deg_call = _make_deg(n_pad, e_pad)
    scat_call = _make_scatter(n_pad, c0, c1, d)

    degp = deg_call(dst_p, ew_p)
    dis = _dis_from_partials(degp.reshape(NW, n_pad))
    dis2 = dis[:n].reshape(n, 1)

    g = _mm_scale(x, W1, dis2)
    for b_i, w_next in ((b1, W2), (b2, W3), (b3, W4), (b4, W5)):
        p = scat_call(g, pk_l, ew_l)
        g = _fused_layer(p, g, dis2, b_i.reshape(1, d), w_next)
    p = scat_call(g, pk_l, ew_l)
    return _finish(p, g, dis2, b5.reshape(1, d))
